# baseline (speedup 1.0000x reference)
import functools

import jax
import jax.numpy as jnp
from jax import lax
from jax.experimental import pallas as pl
from jax.experimental.pallas import tpu as pltpu
from jax.experimental.pallas import tpu_sc as plsc

NC = 2
NS = 16
NW = NC * NS
LANES = 16
CHUNK = 128
BLK = 8


def _wid():
    return lax.axis_index("s") * NC + lax.axis_index("c")


def _mesh():
    return plsc.VectorSubcoreMesh(
        core_axis_name="c", subcore_axis_name="s", num_cores=NC, num_subcores=NS
    )


def _sc_prep(x_r, batch_r, dst_r, table, *, n_pad, pb, d, ech, nch):

    def body(x_ref, b_ref, dst_ref, tab_ref,
             h0_ref, degp_ref, cntp_ref,
             xv, bv, dstv, rows, deg_l, cnt_l, sem):
        cid = lax.axis_index("c")
        sid = lax.axis_index("s")
        wid = sid * NC + cid
        pltpu.sync_copy(x_ref.at[wid], xv)
        pltpu.sync_copy(b_ref.at[wid], bv)
        pltpu.sync_copy(dst_ref.at[wid], dstv)
        o16 = jnp.ones((LANES,), jnp.float32)
        z16 = jnp.zeros((LANES,), jnp.float32)
        lane = lax.iota(jnp.int32, LANES)
        masks = [lane == m for m in range(LANES)]

        def zero_deg(i, carry):
            deg_l[pl.ds(i * LANES, LANES)] = z16
            return carry

        lax.fori_loop(0, n_pad // LANES, zero_deg, 0)
        for i in range(pb // LANES):
            cnt_l[pl.ds(i * LANES, LANES)] = z16

        def deg_step(j, carry):
            for k in range(CHUNK // LANES):
                idx = dstv[j, pl.ds(k * LANES, LANES)]
                for m in range(LANES):
                    plsc.addupdate_scatter(deg_l, [idx], o16, mask=masks[m])
            return carry

        lax.fori_loop(0, ech, deg_step, 0)
        for i in range(nch):
            for k in range(CHUNK // LANES):
                idx = bv[i, pl.ds(k * LANES, LANES)]
                for m in range(LANES):
                    plsc.addupdate_scatter(cnt_l, [idx], o16, mask=masks[m])
        for i in range(nch):
            pltpu.async_copy(tab_ref.at[xv.at[i]], rows, sem).wait()
            pltpu.sync_copy(rows, h0_ref.at[pl.ds(wid * nch * CHUNK + i * CHUNK, CHUNK)])
        pltpu.sync_copy(deg_l, degp_ref.at[wid])
        pltpu.sync_copy(cnt_l, cntp_ref.at[wid])

    f = pl.kernel(
        body,
        out_type=(
            jax.ShapeDtypeStruct((n_pad, d), jnp.float32),
            jax.ShapeDtypeStruct((NW, n_pad), jnp.float32),
            jax.ShapeDtypeStruct((NW, pb), jnp.float32),
        ),
        mesh=_mesh(),
        compiler_params=pltpu.CompilerParams(needs_layout_passes=False),
        scratch_types=(
            pltpu.VMEM((nch, CHUNK), jnp.int32),
            pltpu.VMEM((nch, CHUNK), jnp.int32),
            pltpu.VMEM((ech, CHUNK), jnp.int32),
            pltpu.VMEM((CHUNK, d), jnp.float32),
            pltpu.VMEM((n_pad,), jnp.float32),
            pltpu.VMEM((pb,), jnp.float32),
            pltpu.SemaphoreType.DMA,
        ),
    )
    return f(x_r, batch_r, dst_r, table)


def _sc_propagate(hs, src_r, dst_r, zeros2d, *, agg_r, d, ech):
    rpt = agg_r // NS

    def body(hs_ref, src_ref, dst_ref, z_ref, p_ref,
             srcv, dstv, rows, agg_s, gsem):
        cid = lax.axis_index("c")
        sid = lax.axis_index("s")
        wid = sid * NC + cid
        pltpu.sync_copy(z_ref, agg_s.at[pl.ds(sid * rpt, rpt)])
        plsc.subcore_barrier()

        pltpu.sync_copy(src_ref.at[wid], srcv)
        pltpu.sync_copy(dst_ref.at[wid], dstv)

        def step(j, carry):
            pltpu.async_copy(hs_ref.at[srcv.at[j]], rows, gsem).wait()
            pltpu.sync_copy(rows, agg_s.at[dstv.at[j]], add=True)
            return carry

        lax.fori_loop(0, ech, step, 0)
        plsc.subcore_barrier()

        pltpu.sync_copy(agg_s.at[pl.ds(sid * rpt, rpt)],
                        p_ref.at[cid, pl.ds(sid * rpt, rpt)])

    f = pl.kernel(
        body,
        out_type=jax.ShapeDtypeStruct((NC, agg_r, d), jnp.float32),
        mesh=_mesh(),
        scratch_types=(
            pltpu.VMEM((ech, CHUNK), jnp.int32),
            pltpu.VMEM((ech, CHUNK), jnp.int32),
            pltpu.VMEM((CHUNK, d), jnp.float32),
            pltpu.MemorySpace.VMEM_SHARED((agg_r, d), jnp.float32),
            pltpu.SemaphoreType.DMA,
        ),
    )
    return f(hs, src_r, dst_r, zeros2d)


def _sc_pool(h2, batch_r, zeros2d, *, pb, d, nch):
    ppt = pb // NS

    def body(h_ref, b_ref, z_ref, pp_ref, bv, rows, pool_s):
        cid = lax.axis_index("c")
        sid = lax.axis_index("s")
        wid = sid * NC + cid
        pltpu.sync_copy(b_ref.at[wid], bv)
        pltpu.sync_copy(z_ref.at[pl.ds(0, ppt)], pool_s.at[pl.ds(sid * ppt, ppt)])
        plsc.subcore_barrier()
        for i in range(nch):
            pltpu.sync_copy(h_ref.at[pl.ds(wid * nch * CHUNK + i * CHUNK, CHUNK)], rows)
            pltpu.sync_copy(rows, pool_s.at[bv.at[i]], add=True)
        plsc.subcore_barrier()
        pltpu.sync_copy(pool_s.at[pl.ds(sid * ppt, ppt)],
                        pp_ref.at[cid, pl.ds(sid * ppt, ppt)])

    f = pl.kernel(
        body,
        out_type=jax.ShapeDtypeStruct((NC, pb, d), jnp.float32),
        mesh=_mesh(),
        scratch_types=(
            pltpu.VMEM((nch, CHUNK), jnp.int32),
            pltpu.VMEM((CHUNK, d), jnp.float32),
            pltpu.MemorySpace.VMEM_SHARED((pb, d), jnp.float32),
        ),
    )
    return f(h2, batch_r, zeros2d)


def _tc_prep(degp_c, h0_3, *, n, n_pad, d):
    nb = n_pad // CHUNK

    def body(degp_ref, h0_ref, dis_ref, hs0_ref):
        deg = jnp.sum(degp_ref[...], axis=0)
        rid = (lax.broadcasted_iota(jnp.int32, (nb, CHUNK), 0) * CHUNK
               + lax.broadcasted_iota(jnp.int32, (nb, CHUNK), 1))
        dis = jnp.where(rid < n, lax.rsqrt(jnp.maximum(deg, 1.0)), 0.0)
        dis_ref[...] = dis
        hs0_ref[...] = h0_ref[...] * dis[:, :, None]

    return pl.pallas_call(
        body,
        out_shape=(
            jax.ShapeDtypeStruct((nb, CHUNK), jnp.float32),
            jax.ShapeDtypeStruct((nb, CHUNK, d), jnp.float32),
        ),
        compiler_params=pltpu.CompilerParams(vmem_limit_bytes=128 * 1024 * 1024),
    )(degp_c, h0_3)


def _tc_layer(p3, dis, W, b, *, n, n_pad, agg_r, d, want_hs):
    ab = agg_r // CHUNK

    def body(p_ref, dis_ref, W_ref, b_ref, *out_refs):
        dis3 = dis_ref[...][:ab, :, None]
        agg3 = (p_ref[0] + p_ref[1]) * dis3
        agg = agg3.reshape(agg_r, d)
        z = jnp.dot(agg, W_ref[...], preferred_element_type=jnp.float32,
                    precision=lax.Precision.HIGHEST) + b_ref[...]
        h3 = jnp.maximum(z, 0.0).reshape(ab, CHUNK, d)
        rid = (lax.broadcasted_iota(jnp.int32, (ab, CHUNK, 1), 0) * CHUNK
               + lax.broadcasted_iota(jnp.int32, (ab, CHUNK, 1), 1))
        h3 = jnp.where(rid < n, h3, 0.0)
        tail = jnp.zeros((n_pad - agg_r, d), jnp.float32)
        out_refs[0][:agg_r] = h3.reshape(agg_r, d)
        out_refs[0][agg_r:] = tail
        if want_hs:
            out_refs[1][:agg_r] = (h3 * dis3).reshape(agg_r, d)
            out_refs[1][agg_r:] = tail

    shapes = [jax.ShapeDtypeStruct((n_pad, d), jnp.float32)]
    if want_hs:
        shapes.append(jax.ShapeDtypeStruct((n_pad, d), jnp.float32))
    return pl.pallas_call(
        body,
        out_shape=tuple(shapes),
        compiler_params=pltpu.CompilerParams(vmem_limit_bytes=128 * 1024 * 1024),
    )(p3, dis, W, b)


def _tc_head(pp, cntp_c, global_f, Wl, bl, *, b_graphs, pb, d, g, l):
    def body(pp_ref, cnt_ref, gf_ref, Wl_ref, bl_ref, out_ref):
        pooled_sum = pp_ref[0, :b_graphs] + pp_ref[1, :b_graphs]
        cnt = jnp.sum(cnt_ref[...], axis=0)[:b_graphs]
        pooled = pooled_sum / jnp.maximum(cnt, 1.0)
        Wl_a = Wl_ref[...]
        out = (jnp.dot(pooled, Wl_a[:d], preferred_element_type=jnp.float32,
                       precision=lax.Precision.HIGHEST)
               + jnp.dot(gf_ref[...], Wl_a[d:], preferred_element_type=jnp.float32,
                         precision=lax.Precision.HIGHEST)
               + bl_ref[...])
        out_ref[...] = out

    return pl.pallas_call(
        body,
        out_shape=jax.ShapeDtypeStruct((b_graphs, l), jnp.float32),
        compiler_params=pltpu.CompilerParams(vmem_limit_bytes=128 * 1024 * 1024),
    )(pp, cntp_c, global_f, Wl, bl)


def kernel(x, batch, global_f, edge_index, molecule_embedding, W1, b1, W2, b2, Wl, bl):
    n = x.shape[0]
    e = edge_index.shape[1]
    d = molecule_embedding.shape[1]
    b_graphs = global_f.shape[0]
    g = global_f.shape[1]
    l = Wl.shape[1]

    nch = -(-n // (NW * CHUNK))
    n_pad = NW * nch * CHUNK
    ept = -(-e // NW)
    ech = -(-ept // (CHUNK * BLK)) * BLK
    e_t = ech * CHUNK
    agg_r = -(-(n + 1) // (8 * NS)) * 8 * NS
    pb = (-(-(b_graphs + 1) // (8 * NS))) * 8 * NS
    rpt = agg_r // NS

    x32 = x.astype(jnp.int32)
    batch32 = batch.astype(jnp.int32)
    src = edge_index[0].astype(jnp.int32)
    dst = edge_index[1].astype(jnp.int32)

    x_r = jnp.pad(x32, (0, n_pad - n)).reshape(NW, nch, CHUNK)
    batch_r = jnp.pad(batch32, (0, n_pad - n), constant_values=b_graphs
                      ).reshape(NW, nch, CHUNK)
    src_r = jnp.pad(src.reshape(NW, ept), ((0, 0), (0, e_t - ept))
                    ).reshape(NW, ech, CHUNK)
    dst_r = jnp.pad(dst.reshape(NW, ept), ((0, 0), (0, e_t - ept)),
                    constant_values=n).reshape(NW, ech, CHUNK)
    zeros2d = jnp.zeros((rpt, d), jnp.float32)

    h0, degp, cntp = _sc_prep(x_r, batch_r, dst_r, molecule_embedding,
                              n_pad=n_pad, pb=pb, d=d, ech=ech, nch=nch)
    dis, hs0_3 = _tc_prep(degp.reshape(NW, n_pad // CHUNK, CHUNK),
                          h0.reshape(n_pad // CHUNK, CHUNK, d), n=n, n_pad=n_pad, d=d)
    hs0 = hs0_3.reshape(n_pad, d)
    p1 = _sc_propagate(hs0, src_r, dst_r, zeros2d, agg_r=agg_r, d=d, ech=ech)
    h1, hs1 = _tc_layer(p1.reshape(NC, agg_r // CHUNK, CHUNK, d), dis, W1, b1,
                        n=n, n_pad=n_pad, agg_r=agg_r, d=d, want_hs=True)
    p2 = _sc_propagate(hs1, src_r, dst_r, zeros2d, agg_r=agg_r, d=d, ech=ech)
    (h2,) = _tc_layer(p2.reshape(NC, agg_r // CHUNK, CHUNK, d), dis, W2, b2,
                      n=n, n_pad=n_pad, agg_r=agg_r, d=d, want_hs=False)
    pp = _sc_pool(h2, batch_r, zeros2d, pb=pb, d=d, nch=nch)
    out = _tc_head(pp, cntp.reshape(NW, pb, 1), global_f, Wl, bl,
                   b_graphs=b_graphs, pb=pb, d=d, g=g, l=l)
    return out


# speedup vs baseline: 1.3733x; 1.3733x over previous
import functools

import jax
import jax.numpy as jnp
from jax import lax
from jax.experimental import pallas as pl
from jax.experimental.pallas import tpu as pltpu
from jax.experimental.pallas import tpu_sc as plsc

NC = 2
NS = 16
NW = NC * NS
LANES = 16
CHUNK = 128


def _mesh():
    return plsc.VectorSubcoreMesh(
        core_axis_name="c", subcore_axis_name="s", num_cores=NC, num_subcores=NS
    )


def _sc_prep(x_r, batch_r, dst_r, table, zeros1d, *, n_pad, pb, d, ech, nch):
    deg_seg = n_pad // NS

    def body(x_ref, b_ref, dst_ref, tab_ref, z1_ref,
             h0_ref, degp_ref, cntp_ref,
             xv, bv, dstv, rows, ones_v, deg_s, cnt_s, sem):
        cid = lax.axis_index("c")
        sid = lax.axis_index("s")
        wid = sid * NC + cid
        pltpu.sync_copy(x_ref.at[wid], xv)
        pltpu.sync_copy(b_ref.at[wid], bv)
        pltpu.sync_copy(dst_ref.at[wid], dstv)
        o16 = jnp.ones((LANES,), jnp.float32)
        for k in range(CHUNK // LANES):
            ones_v[pl.ds(k * LANES, LANES)] = o16
        pltpu.sync_copy(z1_ref.at[pl.ds(0, deg_seg)], deg_s.at[pl.ds(sid * deg_seg, deg_seg)])

        @pl.when(sid == 0)
        def _():
            pltpu.sync_copy(z1_ref.at[pl.ds(0, pb)], cnt_s)

        plsc.subcore_barrier()

        def deg_step(j, carry):
            pltpu.sync_copy(ones_v, deg_s.at[dstv.at[j]], add=True)
            return carry

        lax.fori_loop(0, ech, deg_step, 0)
        for i in range(nch):
            pltpu.sync_copy(ones_v, cnt_s.at[bv.at[i]], add=True)
        for i in range(nch):
            pltpu.async_copy(tab_ref.at[xv.at[i]], rows, sem).wait()
            pltpu.sync_copy(rows, h0_ref.at[pl.ds(wid * nch * CHUNK + i * CHUNK, CHUNK)])
        plsc.subcore_barrier()
        pltpu.sync_copy(deg_s.at[pl.ds(sid * deg_seg, deg_seg)],
                        degp_ref.at[cid, pl.ds(sid * deg_seg, deg_seg)])

        @pl.when(sid == 0)
        def _():
            pltpu.sync_copy(cnt_s, cntp_ref.at[cid])

    f = pl.kernel(
        body,
        out_type=(
            jax.ShapeDtypeStruct((n_pad, d), jnp.float32),
            jax.ShapeDtypeStruct((NC, n_pad), jnp.float32),
            jax.ShapeDtypeStruct((NC, pb), jnp.float32),
        ),
        mesh=_mesh(),
        scratch_types=(
            pltpu.VMEM((nch, CHUNK), jnp.int32),
            pltpu.VMEM((nch, CHUNK), jnp.int32),
            pltpu.VMEM((ech, CHUNK), jnp.int32),
            pltpu.VMEM((CHUNK, d), jnp.float32),
            pltpu.VMEM((CHUNK,), jnp.float32),
            pltpu.MemorySpace.VMEM_SHARED((n_pad,), jnp.float32),
            pltpu.MemorySpace.VMEM_SHARED((pb,), jnp.float32),
            pltpu.SemaphoreType.DMA,
        ),
    )
    return f(x_r, batch_r, dst_r, table, zeros1d)


def _sc_propagate(hs, src_r, dst_r, zeros2d, *, agg_r, d, ech):
    rpt = agg_r // NS

    def body(hs_ref, src_ref, dst_ref, z_ref, p_ref,
             srcv, dstv, rows, agg_s, sem):
        cid = lax.axis_index("c")
        sid = lax.axis_index("s")
        wid = sid * NC + cid
        pltpu.sync_copy(src_ref.at[wid], srcv)
        pltpu.sync_copy(dst_ref.at[wid], dstv)
        pltpu.sync_copy(z_ref, agg_s.at[pl.ds(sid * rpt, rpt)])
        plsc.subcore_barrier()

        def step(j, carry):
            pltpu.async_copy(hs_ref.at[srcv.at[j]], rows, sem).wait()
            pltpu.sync_copy(rows, agg_s.at[dstv.at[j]], add=True)
            return carry

        lax.fori_loop(0, ech, step, 0)
        plsc.subcore_barrier()
        pltpu.sync_copy(agg_s.at[pl.ds(sid * rpt, rpt)],
                        p_ref.at[cid, pl.ds(sid * rpt, rpt)])

    f = pl.kernel(
        body,
        out_type=jax.ShapeDtypeStruct((NC, agg_r, d), jnp.float32),
        mesh=_mesh(),
        scratch_types=(
            pltpu.VMEM((ech, CHUNK), jnp.int32),
            pltpu.VMEM((ech, CHUNK), jnp.int32),
            pltpu.VMEM((CHUNK, d), jnp.float32),
            pltpu.MemorySpace.VMEM_SHARED((agg_r, d), jnp.float32),
            pltpu.SemaphoreType.DMA,
        ),
    )
    return f(hs, src_r, dst_r, zeros2d)


def _sc_pool(h2, batch_r, zeros2d, *, pb, d, nch):
    ppt = pb // NS

    def body(h_ref, b_ref, z_ref, pp_ref, bv, rows, pool_s):
        cid = lax.axis_index("c")
        sid = lax.axis_index("s")
        wid = sid * NC + cid
        pltpu.sync_copy(b_ref.at[wid], bv)
        pltpu.sync_copy(z_ref.at[pl.ds(0, ppt)], pool_s.at[pl.ds(sid * ppt, ppt)])
        plsc.subcore_barrier()
        for i in range(nch):
            pltpu.sync_copy(h_ref.at[pl.ds(wid * nch * CHUNK + i * CHUNK, CHUNK)], rows)
            pltpu.sync_copy(rows, pool_s.at[bv.at[i]], add=True)
        plsc.subcore_barrier()
        pltpu.sync_copy(pool_s.at[pl.ds(sid * ppt, ppt)],
                        pp_ref.at[cid, pl.ds(sid * ppt, ppt)])

    f = pl.kernel(
        body,
        out_type=jax.ShapeDtypeStruct((NC, pb, d), jnp.float32),
        mesh=_mesh(),
        scratch_types=(
            pltpu.VMEM((nch, CHUNK), jnp.int32),
            pltpu.VMEM((CHUNK, d), jnp.float32),
            pltpu.MemorySpace.VMEM_SHARED((pb, d), jnp.float32),
        ),
    )
    return f(h2, batch_r, zeros2d)


def _tc_prep(degp_c, h0, *, n, n_pad, d):
    def body(degp_ref, h0_ref, dis_ref, hs0_ref):
        deg = degp_ref[0] + degp_ref[1]
        rid = lax.broadcasted_iota(jnp.int32, (n_pad, 1), 0)
        dis = jnp.where(rid < n, lax.rsqrt(jnp.maximum(deg, 1.0)), 0.0)
        dis_ref[...] = dis
        hs0_ref[...] = h0_ref[...] * dis

    return pl.pallas_call(
        body,
        out_shape=(
            jax.ShapeDtypeStruct((n_pad, 1), jnp.float32),
            jax.ShapeDtypeStruct((n_pad, d), jnp.float32),
        ),
        compiler_params=pltpu.CompilerParams(vmem_limit_bytes=128 * 1024 * 1024),
    )(degp_c, h0)


def _tc_layer(p, dis, W, b, *, n, n_pad, agg_r, d, want_hs):
    def body(p_ref, dis_ref, W_ref, b_ref, *out_refs):
        dis_a = dis_ref[...]
        agg = (p_ref[0] + p_ref[1]) * dis_a[:agg_r]
        z = jnp.dot(agg, W_ref[...], preferred_element_type=jnp.float32,
                    precision=lax.Precision.HIGHEST) + b_ref[...]
        rid = lax.broadcasted_iota(jnp.int32, (agg_r, 1), 0)
        h = jnp.where(rid < n, jnp.maximum(z, 0.0), 0.0)
        tail = jnp.zeros((n_pad - agg_r, d), jnp.float32)
        out_refs[0][:agg_r] = h
        out_refs[0][agg_r:] = tail
        if want_hs:
            out_refs[1][:agg_r] = h * dis_a[:agg_r]
            out_refs[1][agg_r:] = tail

    shapes = [jax.ShapeDtypeStruct((n_pad, d), jnp.float32)]
    if want_hs:
        shapes.append(jax.ShapeDtypeStruct((n_pad, d), jnp.float32))
    return pl.pallas_call(
        body,
        out_shape=tuple(shapes),
        compiler_params=pltpu.CompilerParams(vmem_limit_bytes=128 * 1024 * 1024),
    )(p, dis, W, b)


def _tc_head(pp, cntp_c, global_f, Wl, bl, *, b_graphs, pb, d, g, l):
    def body(pp_ref, cnt_ref, gf_ref, Wl_ref, bl_ref, out_ref):
        pooled_sum = pp_ref[0, :b_graphs] + pp_ref[1, :b_graphs]
        cnt = cnt_ref[0, :b_graphs] + cnt_ref[1, :b_graphs]
        pooled = pooled_sum / jnp.maximum(cnt, 1.0)
        Wl_a = Wl_ref[...]
        out = (jnp.dot(pooled, Wl_a[:d], preferred_element_type=jnp.float32,
                       precision=lax.Precision.HIGHEST)
               + jnp.dot(gf_ref[...], Wl_a[d:], preferred_element_type=jnp.float32,
                         precision=lax.Precision.HIGHEST)
               + bl_ref[...])
        out_ref[...] = out

    return pl.pallas_call(
        body,
        out_shape=jax.ShapeDtypeStruct((b_graphs, l), jnp.float32),
        compiler_params=pltpu.CompilerParams(vmem_limit_bytes=128 * 1024 * 1024),
    )(pp, cntp_c, global_f, Wl, bl)


def kernel(x, batch, global_f, edge_index, molecule_embedding, W1, b1, W2, b2, Wl, bl):
    n = x.shape[0]
    e = edge_index.shape[1]
    d = molecule_embedding.shape[1]
    b_graphs = global_f.shape[0]
    g = global_f.shape[1]
    l = Wl.shape[1]

    nch = -(-n // (NW * CHUNK))
    n_pad = NW * nch * CHUNK
    ept = -(-e // NW)
    ech = -(-ept // CHUNK)
    e_t = ech * CHUNK
    agg_r = -(-(n + 1) // (8 * NS)) * 8 * NS
    pb = (-(-(b_graphs + 1) // (8 * NS))) * 8 * NS
    rpt = agg_r // NS

    x32 = x.astype(jnp.int32)
    batch32 = batch.astype(jnp.int32)
    src = edge_index[0].astype(jnp.int32)
    dst = edge_index[1].astype(jnp.int32)

    x_r = jnp.pad(x32, (0, n_pad - n)).reshape(NW, nch, CHUNK)
    batch_r = jnp.pad(batch32, (0, n_pad - n), constant_values=b_graphs
                      ).reshape(NW, nch, CHUNK)
    src_r = jnp.pad(src.reshape(NW, ept), ((0, 0), (0, e_t - ept))
                    ).reshape(NW, ech, CHUNK)
    dst_r = jnp.pad(dst.reshape(NW, ept), ((0, 0), (0, e_t - ept)),
                    constant_values=n).reshape(NW, ech, CHUNK)
    zeros1d = jnp.zeros((max(n_pad // NS, pb),), jnp.float32)
    zeros2d = jnp.zeros((rpt, d), jnp.float32)

    h0, degp, cntp = _sc_prep(x_r, batch_r, dst_r, molecule_embedding, zeros1d,
                              n_pad=n_pad, pb=pb, d=d, ech=ech, nch=nch)
    dis, hs0 = _tc_prep(degp.reshape(NC, n_pad, 1), h0, n=n, n_pad=n_pad, d=d)
    p1 = _sc_propagate(hs0, src_r, dst_r, zeros2d, agg_r=agg_r, d=d, ech=ech)
    h1, hs1 = _tc_layer(p1, dis, W1, b1, n=n, n_pad=n_pad, agg_r=agg_r, d=d,
                        want_hs=True)
    p2 = _sc_propagate(hs1, src_r, dst_r, zeros2d, agg_r=agg_r, d=d, ech=ech)
    (h2,) = _tc_layer(p2, dis, W2, b2, n=n, n_pad=n_pad, agg_r=agg_r, d=d,
                      want_hs=False)
    pp = _sc_pool(h2, batch_r, zeros2d, pb=pb, d=d, nch=nch)
    out = _tc_head(pp, cntp.reshape(NC, pb, 1), global_f, Wl, bl,
                   b_graphs=b_graphs, pb=pb, d=d, g=g, l=l)
    return out


# speedup vs baseline: 1.3748x; 1.0011x over previous
import functools

import jax
import jax.numpy as jnp
from jax import lax
from jax.experimental import pallas as pl
from jax.experimental.pallas import tpu as pltpu
from jax.experimental.pallas import tpu_sc as plsc

NC = 2
NS = 16
NW = NC * NS
LANES = 16
CHUNK = 128


def _mesh():
    return plsc.VectorSubcoreMesh(
        core_axis_name="c", subcore_axis_name="s", num_cores=NC, num_subcores=NS
    )


def _sc_prep(x_r, batch_r, dst_r, table, *, n_pad, pb, d, ech, nch):
    deg_seg = n_pad // NS

    def body(x_ref, b_ref, dst_ref, tab_ref,
             h0_ref, degp_ref, cntp_ref,
             xv, bv, dstv, rows, deg_l, cnt_l, red, accv, deg_stage, sem):
        cid = lax.axis_index("c")
        sid = lax.axis_index("s")
        wid = sid * NC + cid
        pltpu.sync_copy(x_ref.at[wid], xv)
        pltpu.sync_copy(b_ref.at[wid], bv)
        pltpu.sync_copy(dst_ref.at[wid], dstv)
        o16 = jnp.ones((LANES,), jnp.float32)
        z16 = jnp.zeros((LANES,), jnp.float32)

        def zloop(i, carry):
            deg_l[pl.ds(i * LANES, LANES)] = z16
            return carry

        lax.fori_loop(0, n_pad // LANES, zloop, 0)
        for i in range(pb // LANES):
            cnt_l[pl.ds(i * LANES, LANES)] = z16

        def deg_step(j, carry):
            for k in range(CHUNK // LANES):
                plsc.addupdate_scatter(deg_l, [dstv[j, pl.ds(k * LANES, LANES)]], o16)
            return carry

        lax.fori_loop(0, ech, deg_step, 0)
        for i in range(nch):
            for k in range(CHUNK // LANES):
                plsc.addupdate_scatter(cnt_l, [bv[i, pl.ds(k * LANES, LANES)]], o16)
        for i in range(nch):
            pltpu.async_copy(tab_ref.at[xv.at[i]], rows, sem).wait()
            pltpu.sync_copy(rows, h0_ref.at[pl.ds(wid * nch * CHUNK + i * CHUNK, CHUNK)])
        pltpu.sync_copy(deg_l, deg_stage.at[sid])
        plsc.subcore_barrier()
        pltpu.sync_copy(deg_stage.at[:, pl.ds(sid * deg_seg, deg_seg)], red)

        def radd(i, carry):
            a = red[0, pl.ds(i * LANES, LANES)]
            for s in range(1, NS):
                a = a + red[s, pl.ds(i * LANES, LANES)]
            accv[pl.ds(i * LANES, LANES)] = a
            return carry

        lax.fori_loop(0, deg_seg // LANES, radd, 0)
        pltpu.sync_copy(accv, degp_ref.at[cid, pl.ds(sid * deg_seg, deg_seg)])
        pltpu.sync_copy(cnt_l, cntp_ref.at[wid])

    f = pl.kernel(
        body,
        out_type=(
            jax.ShapeDtypeStruct((n_pad, d), jnp.float32),
            jax.ShapeDtypeStruct((NC, n_pad), jnp.float32),
            jax.ShapeDtypeStruct((NW, pb), jnp.float32),
        ),
        mesh=_mesh(),
        compiler_params=pltpu.CompilerParams(needs_layout_passes=False),
        scratch_types=(
            pltpu.VMEM((nch, CHUNK), jnp.int32),
            pltpu.VMEM((nch, CHUNK), jnp.int32),
            pltpu.VMEM((ech, CHUNK), jnp.int32),
            pltpu.VMEM((CHUNK, d), jnp.float32),
            pltpu.VMEM((n_pad,), jnp.float32),
            pltpu.VMEM((pb,), jnp.float32),
            pltpu.VMEM((NS, n_pad // NS), jnp.float32),
            pltpu.VMEM((n_pad // NS,), jnp.float32),
            pltpu.MemorySpace.VMEM_SHARED((NS, n_pad), jnp.float32),
            pltpu.SemaphoreType.DMA,
        ),
    )
    return f(x_r, batch_r, dst_r, table)


def _sc_propagate(hs, src_r, dst_r, zeros2d, *, agg_r, d, ech):
    rpt = agg_r // NS

    def body(hs_ref, src_ref, dst_ref, z_ref, p_ref,
             srcv, dstv, rows, agg_s, sem):
        cid = lax.axis_index("c")
        sid = lax.axis_index("s")
        wid = sid * NC + cid
        pltpu.sync_copy(src_ref.at[wid], srcv)
        pltpu.sync_copy(dst_ref.at[wid], dstv)
        pltpu.sync_copy(z_ref, agg_s.at[pl.ds(sid * rpt, rpt)])
        plsc.subcore_barrier()

        def step(j, carry):
            pltpu.async_copy(hs_ref.at[srcv.at[j]], rows, sem).wait()
            pltpu.sync_copy(rows, agg_s.at[dstv.at[j]], add=True)
            return carry

        lax.fori_loop(0, ech, step, 0)
        plsc.subcore_barrier()
        pltpu.sync_copy(agg_s.at[pl.ds(sid * rpt, rpt)],
                        p_ref.at[cid, pl.ds(sid * rpt, rpt)])

    f = pl.kernel(
        body,
        out_type=jax.ShapeDtypeStruct((NC, agg_r, d), jnp.float32),
        mesh=_mesh(),
        scratch_types=(
            pltpu.VMEM((ech, CHUNK), jnp.int32),
            pltpu.VMEM((ech, CHUNK), jnp.int32),
            pltpu.VMEM((CHUNK, d), jnp.float32),
            pltpu.MemorySpace.VMEM_SHARED((agg_r, d), jnp.float32),
            pltpu.SemaphoreType.DMA,
        ),
    )
    return f(hs, src_r, dst_r, zeros2d)


def _sc_pool(h2, batch_r, zeros2d, *, pb, d, nch):
    ppt = pb // NS

    def body(h_ref, b_ref, z_ref, pp_ref, bv, rows, pool_s):
        cid = lax.axis_index("c")
        sid = lax.axis_index("s")
        wid = sid * NC + cid
        pltpu.sync_copy(b_ref.at[wid], bv)
        pltpu.sync_copy(z_ref.at[pl.ds(0, ppt)], pool_s.at[pl.ds(sid * ppt, ppt)])
        plsc.subcore_barrier()
        for i in range(nch):
            pltpu.sync_copy(h_ref.at[pl.ds(wid * nch * CHUNK + i * CHUNK, CHUNK)], rows)
            pltpu.sync_copy(rows, pool_s.at[bv.at[i]], add=True)
        plsc.subcore_barrier()
        pltpu.sync_copy(pool_s.at[pl.ds(sid * ppt, ppt)],
                        pp_ref.at[cid, pl.ds(sid * ppt, ppt)])

    f = pl.kernel(
        body,
        out_type=jax.ShapeDtypeStruct((NC, pb, d), jnp.float32),
        mesh=_mesh(),
        scratch_types=(
            pltpu.VMEM((nch, CHUNK), jnp.int32),
            pltpu.VMEM((CHUNK, d), jnp.float32),
            pltpu.MemorySpace.VMEM_SHARED((pb, d), jnp.float32),
        ),
    )
    return f(h2, batch_r, zeros2d)


def _tc_prep(degp_c, h0, *, n, n_pad, d):
    def body(degp_ref, h0_ref, dis_ref, hs0_ref):
        deg = degp_ref[0] + degp_ref[1]
        rid = lax.broadcasted_iota(jnp.int32, (n_pad, 1), 0)
        dis = jnp.where(rid < n, lax.rsqrt(jnp.maximum(deg, 1.0)), 0.0)
        dis_ref[...] = dis
        hs0_ref[...] = h0_ref[...] * dis

    return pl.pallas_call(
        body,
        out_shape=(
            jax.ShapeDtypeStruct((n_pad, 1), jnp.float32),
            jax.ShapeDtypeStruct((n_pad, d), jnp.float32),
        ),
        compiler_params=pltpu.CompilerParams(vmem_limit_bytes=128 * 1024 * 1024),
    )(degp_c, h0)


def _tc_layer(p, dis, W, b, *, n, n_pad, agg_r, d, want_hs):
    def body(p_ref, dis_ref, W_ref, b_ref, *out_refs):
        dis_a = dis_ref[...]
        agg = (p_ref[0] + p_ref[1]) * dis_a[:agg_r]
        z = jnp.dot(agg, W_ref[...], preferred_element_type=jnp.float32,
                    precision=lax.Precision.HIGHEST) + b_ref[...]
        rid = lax.broadcasted_iota(jnp.int32, (agg_r, 1), 0)
        h = jnp.where(rid < n, jnp.maximum(z, 0.0), 0.0)
        tail = jnp.zeros((n_pad - agg_r, d), jnp.float32)
        out_refs[0][:agg_r] = h
        out_refs[0][agg_r:] = tail
        if want_hs:
            out_refs[1][:agg_r] = h * dis_a[:agg_r]
            out_refs[1][agg_r:] = tail

    shapes = [jax.ShapeDtypeStruct((n_pad, d), jnp.float32)]
    if want_hs:
        shapes.append(jax.ShapeDtypeStruct((n_pad, d), jnp.float32))
    return pl.pallas_call(
        body,
        out_shape=tuple(shapes),
        compiler_params=pltpu.CompilerParams(vmem_limit_bytes=128 * 1024 * 1024),
    )(p, dis, W, b)


def _tc_head(pp, cntp_c, global_f, Wl, bl, *, b_graphs, pb, d, g, l):
    def body(pp_ref, cnt_ref, gf_ref, Wl_ref, bl_ref, out_ref):
        pooled_sum = pp_ref[0, :b_graphs] + pp_ref[1, :b_graphs]
        cnt = jnp.sum(cnt_ref[...], axis=0)[:b_graphs]
        pooled = pooled_sum / jnp.maximum(cnt, 1.0)
        Wl_a = Wl_ref[...]
        out = (jnp.dot(pooled, Wl_a[:d], preferred_element_type=jnp.float32,
                       precision=lax.Precision.HIGHEST)
               + jnp.dot(gf_ref[...], Wl_a[d:], preferred_element_type=jnp.float32,
                         precision=lax.Precision.HIGHEST)
               + bl_ref[...])
        out_ref[...] = out

    return pl.pallas_call(
        body,
        out_shape=jax.ShapeDtypeStruct((b_graphs, l), jnp.float32),
        compiler_params=pltpu.CompilerParams(vmem_limit_bytes=128 * 1024 * 1024),
    )(pp, cntp_c, global_f, Wl, bl)


def kernel(x, batch, global_f, edge_index, molecule_embedding, W1, b1, W2, b2, Wl, bl):
    n = x.shape[0]
    e = edge_index.shape[1]
    d = molecule_embedding.shape[1]
    b_graphs = global_f.shape[0]
    g = global_f.shape[1]
    l = Wl.shape[1]

    nch = -(-n // (NW * CHUNK))
    n_pad = NW * nch * CHUNK
    ept = -(-e // NW)
    ech = -(-ept // CHUNK)
    e_t = ech * CHUNK
    agg_r = -(-(n + 1) // (8 * NS)) * 8 * NS
    pb = (-(-(b_graphs + 1) // (8 * NS))) * 8 * NS
    rpt = agg_r // NS

    x32 = x.astype(jnp.int32)
    batch32 = batch.astype(jnp.int32)
    src = edge_index[0].astype(jnp.int32)
    dst = edge_index[1].astype(jnp.int32)

    x_r = jnp.pad(x32, (0, n_pad - n)).reshape(NW, nch, CHUNK)
    batch_r = jnp.pad(batch32, (0, n_pad - n), constant_values=b_graphs
                      ).reshape(NW, nch, CHUNK)
    src_r = jnp.pad(src.reshape(NW, ept), ((0, 0), (0, e_t - ept))
                    ).reshape(NW, ech, CHUNK)
    dst_r = jnp.pad(dst.reshape(NW, ept), ((0, 0), (0, e_t - ept)),
                    constant_values=n).reshape(NW, ech, CHUNK)
    zeros2d = jnp.zeros((rpt, d), jnp.float32)

    h0, degp, cntp = _sc_prep(x_r, batch_r, dst_r, molecule_embedding,
                              n_pad=n_pad, pb=pb, d=d, ech=ech, nch=nch)
    dis, hs0 = _tc_prep(degp.reshape(NC, n_pad, 1), h0, n=n, n_pad=n_pad, d=d)
    p1 = _sc_propagate(hs0, src_r, dst_r, zeros2d, agg_r=agg_r, d=d, ech=ech)
    h1, hs1 = _tc_layer(p1, dis, W1, b1, n=n, n_pad=n_pad, agg_r=agg_r, d=d,
                        want_hs=True)
    p2 = _sc_propagate(hs1, src_r, dst_r, zeros2d, agg_r=agg_r, d=d, ech=ech)
    (h2,) = _tc_layer(p2, dis, W2, b2, n=n, n_pad=n_pad, agg_r=agg_r, d=d,
                      want_hs=False)
    pp = _sc_pool(h2, batch_r, zeros2d, pb=pb, d=d, nch=nch)
    out = _tc_head(pp, cntp.reshape(NW, pb, 1), global_f, Wl, bl,
                   b_graphs=b_graphs, pb=pb, d=d, g=g, l=l)
    return out
